# baseline (speedup 1.0000x reference)
import functools

import jax
import jax.numpy as jnp
from jax import lax
from jax.experimental import pallas as pl
from jax.experimental.pallas import tpu as pltpu
from jax.experimental.pallas import tpu_sc as plsc

N = 10000
C = 128
E2 = 320000
NC, NS = 2, 16
H1 = 256
EPS = 1e-5

ROWS_PER_TILE = E2 // NS
SCH = 80
NCHUNK = ROWS_PER_TILE // SCH
HALF = E2 // 2
HPT = HALF // NS
NCHUNK2 = HPT // SCH
NPT = 624
NREM = N - NS * NPT
NROFF = NS * NPT


def _scatter_body(edge_hbm, idx2_hbm, zeros_nc, ones_nc,
                  st_out, deg_out,
                  acc, ix0, ix1, rw0, rw1, ones_v,
                  sl0, sl1, sc0, sc1):
    ci = lax.axis_index("c")
    s = lax.axis_index("s")
    nb = s * NPT

    def zero_acc():
        pltpu.sync_copy(zeros_nc.at[pl.ds(nb, NPT)], acc.at[pl.ds(nb, NPT)])

        @pl.when(s == NS - 1)
        def _():
            pltpu.sync_copy(zeros_nc.at[pl.ds(NROFF, NREM)],
                            acc.at[pl.ds(NROFF, NREM)])

    def write_acc(out_hbm):
        pltpu.sync_copy(acc.at[pl.ds(nb, NPT)], out_hbm.at[ci, pl.ds(nb, NPT)])

        @pl.when(s == NS - 1)
        def _():
            pltpu.sync_copy(acc.at[pl.ds(NROFF, NREM)],
                            out_hbm.at[ci, pl.ds(NROFF, NREM)])

    zero_acc()
    pltpu.sync_copy(ones_nc, ones_v)
    plsc.subcore_barrier()

    slots = ((ix0, rw0, sl0, sc0), (ix1, rw1, sl1, sc1))

    def p1_load(k, b):
        ix, rw, sl, _ = slots[b]
        base = s * ROWS_PER_TILE + k * SCH
        pltpu.async_copy(idx2_hbm.at[pl.ds(ci * E2 + base, SCH)], ix, sl)
        pltpu.async_copy(edge_hbm.at[pl.ds(base, SCH)], rw, sl)

    def p1_load_wait(b):
        ix, rw, sl, _ = slots[b]
        pltpu.make_async_copy(idx2_hbm.at[pl.ds(0, SCH)], ix, sl).wait()
        pltpu.make_async_copy(edge_hbm.at[pl.ds(0, SCH)], rw, sl).wait()

    def scat(b, src):
        ix, rw, _, sc = slots[b]
        pltpu.async_copy(src if src is not None else rw, acc.at[ix], sc,
                         add=True)

    def scat_wait(b, src):
        ix, rw, _, sc = slots[b]
        pltpu.make_async_copy(src if src is not None else rw, acc.at[ix],
                              sc).wait()

    p1_load(0, 0)

    def p1_pair(j, carry):
        k0 = 2 * j
        p1_load_wait(0)
        scat(0, None)

        @pl.when(j > 0)
        def _():
            scat_wait(1, None)

        p1_load(k0 + 1, 1)
        p1_load_wait(1)
        scat(1, None)
        scat_wait(0, None)

        @pl.when(j < NCHUNK // 2 - 1)
        def _():
            p1_load(k0 + 2, 0)

        return carry

    lax.fori_loop(0, NCHUNK // 2, p1_pair, 0)
    scat_wait(1, None)
    plsc.subcore_barrier()
    write_acc(st_out)
    zero_acc()
    plsc.subcore_barrier()

    def p2_load(k, b):
        ix, _, sl, _ = slots[b]
        base = ci * HALF + s * HPT + k * SCH
        pltpu.async_copy(idx2_hbm.at[pl.ds(ci * E2 + base, SCH)], ix, sl)

    def p2_load_wait(b):
        ix, _, sl, _ = slots[b]
        pltpu.make_async_copy(idx2_hbm.at[pl.ds(0, SCH)], ix, sl).wait()

    p2_load(0, 0)
    p2_load_wait(0)
    scat(0, ones_v)
    p2_load(1, 1)

    def p2_pair(j, carry):
        k0 = 2 * j + 1
        p2_load_wait(1)
        scat(1, ones_v)
        scat_wait(0, ones_v)
        p2_load(k0 + 1, 0)
        p2_load_wait(0)
        scat(0, ones_v)
        scat_wait(1, ones_v)

        @pl.when(j < (NCHUNK2 - 1) // 2 - 1)
        def _():
            p2_load(k0 + 2, 1)

        return carry

    lax.fori_loop(0, (NCHUNK2 - 1) // 2, p2_pair, 0)
    scat_wait(0, ones_v)
    plsc.subcore_barrier()
    write_acc(deg_out)


_scatter = pl.kernel(
    _scatter_body,
    out_type=[jax.ShapeDtypeStruct((NC, N, C), jnp.float32),
              jax.ShapeDtypeStruct((NC, N, C), jnp.float32)],
    mesh=plsc.VectorSubcoreMesh(core_axis_name="c", subcore_axis_name="s",
                                num_cores=NC, num_subcores=NS),
    scratch_types=[pltpu.VMEM_SHARED((N, C), jnp.float32),
                   pltpu.VMEM((SCH,), jnp.int32),
                   pltpu.VMEM((SCH,), jnp.int32),
                   pltpu.VMEM((SCH, C), jnp.float32),
                   pltpu.VMEM((SCH, C), jnp.float32),
                   pltpu.VMEM((SCH, C), jnp.float32),
                   pltpu.SemaphoreType.DMA,
                   pltpu.SemaphoreType.DMA,
                   pltpu.SemaphoreType.DMA,
                   pltpu.SemaphoreType.DMA],
)

ROWS_PER_W = E2 // (NC * NS)
GCH = 40
NG = ROWS_PER_W // GCH


def _gather_body(pq_hbm, idxs_hbm, r_out,
                 is0, is1, pb0, ob0, pb1, ob1,
                 si0, si1, sg0, sg1, ss0, ss1):
    ci = lax.axis_index("c")
    s = lax.axis_index("s")
    w = s * NC + ci
    rbase = w * ROWS_PER_W

    islot = ((is0, si0), (is1, si1))
    gslot = ((pb0, ob0, sg0, ss0), (pb1, ob1, sg1, ss1))

    def issue_idx(k, b):
        isv, si = islot[b]
        pltpu.async_copy(idxs_hbm.at[pl.ds(rbase + k * GCH, GCH)], isv, si)

    def wait_idx(b):
        isv, si = islot[b]
        pltpu.make_async_copy(idxs_hbm.at[pl.ds(0, GCH)], isv, si).wait()

    def issue_gather(b):
        isv, _ = islot[b]
        pb, ob, sg, ss = gslot[b]
        pltpu.async_copy(pq_hbm.at[isv], pb, sg)

    def wait_gather(b):
        isv, _ = islot[b]
        pb, ob, sg, ss = gslot[b]
        pltpu.make_async_copy(pq_hbm.at[isv], pb, sg).wait()

    def do_add(b):
        pb, ob, sg, ss = gslot[b]

        def addpair(i, c2):
            ra = 2 * i
            rb = 2 * i + 1
            for j in range(H1 // 16):
                sl = pl.ds(j * 16, 16)
                so = pl.ds(H1 + j * 16, 16)
                ob[ra, sl] = pb[ra, sl] + pb[rb, so]
                ob[rb, sl] = pb[rb, sl] + pb[ra, so]
            return c2

        lax.fori_loop(0, GCH // 2, addpair, 0)

    def issue_store(k, b):
        pb, ob, sg, ss = gslot[b]
        pltpu.async_copy(ob, r_out.at[pl.ds(rbase + k * GCH, GCH)], ss)

    def wait_store(b):
        pb, ob, sg, ss = gslot[b]
        pltpu.make_async_copy(ob, r_out.at[pl.ds(rbase, GCH)], ss).wait()

    issue_idx(0, 0)
    wait_idx(0)
    issue_gather(0)
    issue_idx(1, 1)

    def pair(j, carry):
        k0 = 2 * j
        wait_idx(1)
        issue_gather(1)
        wait_gather(0)

        @pl.when(j > 0)
        def _():
            wait_store(0)

        do_add(0)
        issue_store(k0, 0)

        @pl.when(k0 + 2 < NG)
        def _():
            issue_idx(k0 + 2, 0)
            wait_idx(0)
            issue_gather(0)

        wait_gather(1)

        @pl.when(j > 0)
        def _():
            wait_store(1)

        do_add(1)
        issue_store(k0 + 1, 1)

        @pl.when(k0 + 3 < NG)
        def _():
            issue_idx(k0 + 3, 1)

        return carry

    lax.fori_loop(0, NG // 2, pair, 0)
    wait_store(0)
    wait_store(1)


_gather = pl.kernel(
    _gather_body,
    out_type=jax.ShapeDtypeStruct((E2, H1), jnp.float32),
    mesh=plsc.VectorSubcoreMesh(core_axis_name="c", subcore_axis_name="s",
                                num_cores=NC, num_subcores=NS),
    scratch_types=[pltpu.VMEM((GCH,), jnp.int32),
                   pltpu.VMEM((GCH,), jnp.int32),
                   pltpu.VMEM((GCH, 2 * H1), jnp.float32),
                   pltpu.VMEM((GCH, H1), jnp.float32),
                   pltpu.VMEM((GCH, 2 * H1), jnp.float32),
                   pltpu.VMEM((GCH, H1), jnp.float32),
                   pltpu.SemaphoreType.DMA,
                   pltpu.SemaphoreType.DMA,
                   pltpu.SemaphoreType.DMA,
                   pltpu.SemaphoreType.DMA,
                   pltpu.SemaphoreType.DMA,
                   pltpu.SemaphoreType.DMA],
)


def _pq_body(w1_ref, s_ref, t_ref, d0_ref, d1_ref, pq_ref, b0_ref):
    w = w1_ref[...]
    a0 = w[0:C]
    a1 = w[C:2 * C]
    a2 = w[2 * C:3 * C]
    a3 = w[3 * C:4 * C]
    a4 = w[4 * C:5 * C]
    b1 = a1 + a2 + a3 + a4
    b2 = a2 + a4
    b3 = a3 + a4
    sv = s_ref[...]
    d = d0_ref[...] + d1_ref[...]
    p = (jnp.dot(sv * d, b1, preferred_element_type=jnp.float32)
         + jnp.dot(t_ref[...], b2, preferred_element_type=jnp.float32))
    q = jnp.dot(sv, b3, preferred_element_type=jnp.float32)
    pq_ref[...] = jnp.concatenate([p, q], axis=1)
    b0_ref[...] = a0 + a4


def _pq(w1, s, t, d0, d1):
    return pl.pallas_call(
        _pq_body,
        out_shape=[jax.ShapeDtypeStruct((N, 2 * H1), jnp.float32),
                   jax.ShapeDtypeStruct((C, H1), jnp.float32)],
    )(w1, s, t, d0, d1)


BM = 2560
NBLK = E2 // BM


def _p1_body(x_ref, r_ref, b0_ref, h_ref, st_ref):
    h = jnp.dot(x_ref[...], b0_ref[...],
                preferred_element_type=jnp.float32) + r_ref[...]
    h_ref[...] = h.astype(jnp.bfloat16)
    st = jnp.concatenate([jnp.sum(h, 0, keepdims=True),
                          jnp.sum(h * h, 0, keepdims=True)], axis=0)

    @pl.when(pl.program_id(0) == 0)
    def _():
        st_ref[...] = st

    @pl.when(pl.program_id(0) != 0)
    def _():
        st_ref[...] += st


def _pass1(x, rm, b0):
    return pl.pallas_call(
        _p1_body,
        grid=(NBLK,),
        in_specs=[pl.BlockSpec((BM, C), lambda i: (i, 0)),
                  pl.BlockSpec((BM, H1), lambda i: (i, 0)),
                  pl.BlockSpec((C, H1), lambda i: (0, 0))],
        out_specs=[pl.BlockSpec((BM, H1), lambda i: (i, 0)),
                   pl.BlockSpec((2, H1), lambda i: (0, 0))],
        out_shape=[jax.ShapeDtypeStruct((E2, H1), jnp.bfloat16),
                   jax.ShapeDtypeStruct((2, H1), jnp.float32)],
    )(x, rm, b0)


def _p2_body(h_ref, st_ref, g_ref, b_ref, w2_ref, h2_ref, st2_ref):
    st = st_ref[...]
    mu = st[0:1] * (1.0 / E2)
    ex2 = st[1:2] * (1.0 / E2)
    inv = lax.rsqrt(ex2 - mu * mu + EPS)
    hv = h_ref[...].astype(jnp.float32)
    a = jnp.maximum((hv - mu) * (inv * g_ref[...]) + b_ref[...], 0.0)
    h2 = jnp.dot(a, w2_ref[...], preferred_element_type=jnp.float32)
    h2_ref[...] = h2.astype(jnp.bfloat16)
    st2 = jnp.concatenate([jnp.sum(h2, 0, keepdims=True),
                           jnp.sum(h2 * h2, 0, keepdims=True)], axis=0)

    @pl.when(pl.program_id(0) == 0)
    def _():
        st2_ref[...] = st2

    @pl.when(pl.program_id(0) != 0)
    def _():
        st2_ref[...] += st2


def _pass2(h, st, g1, b1, w2):
    return pl.pallas_call(
        _p2_body,
        grid=(NBLK,),
        in_specs=[pl.BlockSpec((BM, H1), lambda i: (i, 0)),
                  pl.BlockSpec((2, H1), lambda i: (0, 0)),
                  pl.BlockSpec((1, H1), lambda i: (0, 0)),
                  pl.BlockSpec((1, H1), lambda i: (0, 0)),
                  pl.BlockSpec((H1, C), lambda i: (0, 0))],
        out_specs=[pl.BlockSpec((BM, C), lambda i: (i, 0)),
                   pl.BlockSpec((2, C), lambda i: (0, 0))],
        out_shape=[jax.ShapeDtypeStruct((E2, C), jnp.bfloat16),
                   jax.ShapeDtypeStruct((2, C), jnp.float32)],
    )(h, st, g1, b1, w2)


def _p3_body(h2_ref, st2_ref, g_ref, b_ref, o_ref):
    st = st2_ref[...]
    mu = st[0:1] * (1.0 / E2)
    ex2 = st[1:2] * (1.0 / E2)
    inv = lax.rsqrt(ex2 - mu * mu + EPS)
    h2v = h2_ref[...].astype(jnp.float32)
    o_ref[...] = jnp.maximum((h2v - mu) * (inv * g_ref[...]) + b_ref[...], 0.0)


def _pass3(h2, st2, g2, b2):
    return pl.pallas_call(
        _p3_body,
        grid=(NBLK,),
        in_specs=[pl.BlockSpec((BM, C), lambda i: (i, 0)),
                  pl.BlockSpec((2, C), lambda i: (0, 0)),
                  pl.BlockSpec((1, C), lambda i: (0, 0)),
                  pl.BlockSpec((1, C), lambda i: (0, 0))],
        out_specs=pl.BlockSpec((BM, C), lambda i: (i, 0)),
        out_shape=jax.ShapeDtypeStruct((E2, C), jnp.float32),
    )(h2, st2, g2, b2)


def kernel(edge_rep, edge_index, W1, gamma1, beta1, W2, gamma2, beta2):
    src = edge_index[0]
    dst = edge_index[1]
    idx_own = jnp.stack([src, dst], axis=1).reshape(-1)
    idx_oth = jnp.stack([dst, src], axis=1).reshape(-1)
    idx2 = jnp.concatenate([idx_own, idx_oth])
    zeros_nc = jnp.zeros((N, C), jnp.float32)
    ones_nc = jnp.ones((SCH, C), jnp.float32)

    st, degw = _scatter(edge_rep, idx2, zeros_nc, ones_nc)
    s_acc = st[0]
    t_acc = st[1]

    pq, b0 = _pq(W1, s_acc, t_acc, degw[0, :, :1], degw[1, :, :1])
    rm = _gather(pq, idx_own)
    h, st1 = _pass1(edge_rep, rm, b0)
    h2, st2 = _pass2(h, st1, gamma1.reshape(1, H1), beta1.reshape(1, H1), W2)
    return _pass3(h2, st2, gamma2.reshape(1, C), beta2.reshape(1, C))


# speedup vs baseline: 1.3641x; 1.3641x over previous
import functools

import jax
import jax.numpy as jnp
from jax import lax
from jax.experimental import pallas as pl
from jax.experimental.pallas import tpu as pltpu
from jax.experimental.pallas import tpu_sc as plsc

N = 10000
C = 128
E2 = 320000
NC, NS = 2, 16
H1 = 256
EPS = 1e-5

ROWS_PER_TILE = E2 // NS
SCH = 80
NCHUNK = ROWS_PER_TILE // SCH
HALF = E2 // 2
HPT = HALF // NS
NCHUNK2 = HPT // SCH
NPT = 624
NREM = N - NS * NPT
NROFF = NS * NPT


def _scatter_body(edge_hbm, idx2_hbm, zeros_nc, ones_nc,
                  st_out, deg_out,
                  acc, ix0, ix1, rw0, rw1, ones_v,
                  sl0, sl1, sc0, sc1):
    ci = lax.axis_index("c")
    s = lax.axis_index("s")
    nb = s * NPT

    def zero_acc():
        pltpu.sync_copy(zeros_nc.at[pl.ds(nb, NPT)], acc.at[pl.ds(nb, NPT)])

        @pl.when(s == NS - 1)
        def _():
            pltpu.sync_copy(zeros_nc.at[pl.ds(NROFF, NREM)],
                            acc.at[pl.ds(NROFF, NREM)])

    def write_acc(out_hbm):
        pltpu.sync_copy(acc.at[pl.ds(nb, NPT)], out_hbm.at[ci, pl.ds(nb, NPT)])

        @pl.when(s == NS - 1)
        def _():
            pltpu.sync_copy(acc.at[pl.ds(NROFF, NREM)],
                            out_hbm.at[ci, pl.ds(NROFF, NREM)])

    zero_acc()
    pltpu.sync_copy(ones_nc, ones_v)
    plsc.subcore_barrier()

    slots = ((ix0, rw0, sl0, sc0), (ix1, rw1, sl1, sc1))

    def p1_load(k, b):
        ix, rw, sl, _ = slots[b]
        base = s * ROWS_PER_TILE + k * SCH
        pltpu.async_copy(idx2_hbm.at[pl.ds(ci * E2 + base, SCH)], ix, sl)
        pltpu.async_copy(edge_hbm.at[pl.ds(base, SCH)], rw, sl)

    def p1_load_wait(b):
        ix, rw, sl, _ = slots[b]
        pltpu.make_async_copy(idx2_hbm.at[pl.ds(0, SCH)], ix, sl).wait()
        pltpu.make_async_copy(edge_hbm.at[pl.ds(0, SCH)], rw, sl).wait()

    def scat(b, src):
        ix, rw, _, sc = slots[b]
        pltpu.async_copy(src if src is not None else rw, acc.at[ix], sc,
                         add=True)

    def scat_wait(b, src):
        ix, rw, _, sc = slots[b]
        pltpu.make_async_copy(src if src is not None else rw, acc.at[ix],
                              sc).wait()

    p1_load(0, 0)

    def p1_pair(j, carry):
        k0 = 2 * j
        p1_load_wait(0)
        scat(0, None)

        @pl.when(j > 0)
        def _():
            scat_wait(1, None)

        p1_load(k0 + 1, 1)
        p1_load_wait(1)
        scat(1, None)
        scat_wait(0, None)

        @pl.when(j < NCHUNK // 2 - 1)
        def _():
            p1_load(k0 + 2, 0)

        return carry

    lax.fori_loop(0, NCHUNK // 2, p1_pair, 0)
    scat_wait(1, None)
    plsc.subcore_barrier()
    write_acc(st_out)
    zero_acc()
    plsc.subcore_barrier()

    def p2_load(k, b):
        ix, _, sl, _ = slots[b]
        base = ci * HALF + s * HPT + k * SCH
        pltpu.async_copy(idx2_hbm.at[pl.ds(ci * E2 + base, SCH)], ix, sl)

    def p2_load_wait(b):
        ix, _, sl, _ = slots[b]
        pltpu.make_async_copy(idx2_hbm.at[pl.ds(0, SCH)], ix, sl).wait()

    p2_load(0, 0)
    p2_load_wait(0)
    scat(0, ones_v)
    p2_load(1, 1)

    def p2_pair(j, carry):
        k0 = 2 * j + 1
        p2_load_wait(1)
        scat(1, ones_v)
        scat_wait(0, ones_v)
        p2_load(k0 + 1, 0)
        p2_load_wait(0)
        scat(0, ones_v)
        scat_wait(1, ones_v)

        @pl.when(j < (NCHUNK2 - 1) // 2 - 1)
        def _():
            p2_load(k0 + 2, 1)

        return carry

    lax.fori_loop(0, (NCHUNK2 - 1) // 2, p2_pair, 0)
    scat_wait(0, ones_v)
    plsc.subcore_barrier()
    write_acc(deg_out)


_scatter = pl.kernel(
    _scatter_body,
    out_type=[jax.ShapeDtypeStruct((NC, N, C), jnp.float32),
              jax.ShapeDtypeStruct((NC, N, C), jnp.float32)],
    mesh=plsc.VectorSubcoreMesh(core_axis_name="c", subcore_axis_name="s",
                                num_cores=NC, num_subcores=NS),
    scratch_types=[pltpu.VMEM_SHARED((N, C), jnp.float32),
                   pltpu.VMEM((SCH,), jnp.int32),
                   pltpu.VMEM((SCH,), jnp.int32),
                   pltpu.VMEM((SCH, C), jnp.float32),
                   pltpu.VMEM((SCH, C), jnp.float32),
                   pltpu.VMEM((SCH, C), jnp.float32),
                   pltpu.SemaphoreType.DMA,
                   pltpu.SemaphoreType.DMA,
                   pltpu.SemaphoreType.DMA,
                   pltpu.SemaphoreType.DMA],
)

ROWS_PER_W = E2 // (NC * NS)
GCH = 40
NG = ROWS_PER_W // GCH


def _gather_body(p_hbm, q_hbm, idxs_hbm, idxt_hbm, r_out,
                 is0, it0, is1, it1, pb0, qb0, ob0, pb1, qb1, ob1,
                 si0, si1, sg0, sg1, ss0, ss1):
    ci = lax.axis_index("c")
    s = lax.axis_index("s")
    w = s * NC + ci
    rbase = w * ROWS_PER_W

    islot = ((is0, it0, si0), (is1, it1, si1))
    gslot = ((pb0, qb0, ob0, sg0, ss0), (pb1, qb1, ob1, sg1, ss1))

    def issue_idx(k, b):
        isv, itv, si = islot[b]
        pltpu.async_copy(idxs_hbm.at[pl.ds(rbase + k * GCH, GCH)], isv, si)
        pltpu.async_copy(idxt_hbm.at[pl.ds(rbase + k * GCH, GCH)], itv, si)

    def wait_idx(b):
        isv, itv, si = islot[b]
        pltpu.make_async_copy(idxs_hbm.at[pl.ds(0, GCH)], isv, si).wait()
        pltpu.make_async_copy(idxt_hbm.at[pl.ds(0, GCH)], itv, si).wait()

    def issue_gather(b):
        isv, itv, si = islot[b]
        pb, qb, ob, sg, ss = gslot[b]
        pltpu.async_copy(p_hbm.at[isv], pb, sg)
        pltpu.async_copy(q_hbm.at[itv], qb, sg)

    def wait_gather(b):
        isv, itv, si = islot[b]
        pb, qb, ob, sg, ss = gslot[b]
        pltpu.make_async_copy(p_hbm.at[isv], pb, sg).wait()
        pltpu.make_async_copy(q_hbm.at[itv], qb, sg).wait()

    def do_add(b):
        pb, qb, ob, sg, ss = gslot[b]

        def addrow(i, c2):
            for j in range(H1 // 16):
                sl = pl.ds(j * 16, 16)
                ob[i, sl] = pb[i, sl] + qb[i, sl]
            return c2

        lax.fori_loop(0, GCH, addrow, 0)

    def issue_store(k, b):
        pb, qb, ob, sg, ss = gslot[b]
        pltpu.async_copy(ob, r_out.at[pl.ds(rbase + k * GCH, GCH)], ss)

    def wait_store(b):
        pb, qb, ob, sg, ss = gslot[b]
        pltpu.make_async_copy(ob, r_out.at[pl.ds(rbase, GCH)], ss).wait()

    issue_idx(0, 0)
    wait_idx(0)
    issue_gather(0)
    issue_idx(1, 1)

    def pair(j, carry):
        k0 = 2 * j
        wait_idx(1)
        issue_gather(1)
        wait_gather(0)

        @pl.when(j > 0)
        def _():
            wait_store(0)

        do_add(0)
        issue_store(k0, 0)

        @pl.when(k0 + 2 < NG)
        def _():
            issue_idx(k0 + 2, 0)
            wait_idx(0)
            issue_gather(0)

        wait_gather(1)

        @pl.when(j > 0)
        def _():
            wait_store(1)

        do_add(1)
        issue_store(k0 + 1, 1)

        @pl.when(k0 + 3 < NG)
        def _():
            issue_idx(k0 + 3, 1)

        return carry

    lax.fori_loop(0, NG // 2, pair, 0)
    wait_store(0)
    wait_store(1)


_gather = pl.kernel(
    _gather_body,
    out_type=jax.ShapeDtypeStruct((E2, H1), jnp.float32),
    mesh=plsc.VectorSubcoreMesh(core_axis_name="c", subcore_axis_name="s",
                                num_cores=NC, num_subcores=NS),
    scratch_types=[pltpu.VMEM((GCH,), jnp.int32),
                   pltpu.VMEM((GCH,), jnp.int32),
                   pltpu.VMEM((GCH,), jnp.int32),
                   pltpu.VMEM((GCH,), jnp.int32),
                   pltpu.VMEM((GCH, H1), jnp.float32),
                   pltpu.VMEM((GCH, H1), jnp.float32),
                   pltpu.VMEM((GCH, H1), jnp.float32),
                   pltpu.VMEM((GCH, H1), jnp.float32),
                   pltpu.VMEM((GCH, H1), jnp.float32),
                   pltpu.VMEM((GCH, H1), jnp.float32),
                   pltpu.SemaphoreType.DMA,
                   pltpu.SemaphoreType.DMA,
                   pltpu.SemaphoreType.DMA,
                   pltpu.SemaphoreType.DMA,
                   pltpu.SemaphoreType.DMA,
                   pltpu.SemaphoreType.DMA],
)


def _pq_body(w1_ref, s_ref, t_ref, d0_ref, d1_ref, p_ref, q_ref, b0_ref):
    w = w1_ref[...]
    a0 = w[0:C]
    a1 = w[C:2 * C]
    a2 = w[2 * C:3 * C]
    a3 = w[3 * C:4 * C]
    a4 = w[4 * C:5 * C]
    b1 = a1 + a2 + a3 + a4
    b2 = a2 + a4
    b3 = a3 + a4
    sv = s_ref[...]
    d = d0_ref[...] + d1_ref[...]
    p_ref[...] = (jnp.dot(sv * d, b1, preferred_element_type=jnp.float32)
                  + jnp.dot(t_ref[...], b2, preferred_element_type=jnp.float32))
    q_ref[...] = jnp.dot(sv, b3, preferred_element_type=jnp.float32)
    b0_ref[...] = a0 + a4


def _pq(w1, s, t, d0, d1):
    return pl.pallas_call(
        _pq_body,
        out_shape=[jax.ShapeDtypeStruct((N, H1), jnp.float32),
                   jax.ShapeDtypeStruct((N, H1), jnp.float32),
                   jax.ShapeDtypeStruct((C, H1), jnp.float32)],
    )(w1, s, t, d0, d1)


BM = 2560
NBLK = E2 // BM


def _p1_body(x_ref, r_ref, b0_ref, h_ref, st_ref):
    h = jnp.dot(x_ref[...], b0_ref[...],
                preferred_element_type=jnp.float32) + r_ref[...]
    h_ref[...] = h.astype(jnp.bfloat16)
    st = jnp.concatenate([jnp.sum(h, 0, keepdims=True),
                          jnp.sum(h * h, 0, keepdims=True)], axis=0)

    @pl.when(pl.program_id(0) == 0)
    def _():
        st_ref[...] = st

    @pl.when(pl.program_id(0) != 0)
    def _():
        st_ref[...] += st


def _pass1(x, rm, b0):
    return pl.pallas_call(
        _p1_body,
        grid=(NBLK,),
        in_specs=[pl.BlockSpec((BM, C), lambda i: (i, 0)),
                  pl.BlockSpec((BM, H1), lambda i: (i, 0)),
                  pl.BlockSpec((C, H1), lambda i: (0, 0))],
        out_specs=[pl.BlockSpec((BM, H1), lambda i: (i, 0)),
                   pl.BlockSpec((2, H1), lambda i: (0, 0))],
        out_shape=[jax.ShapeDtypeStruct((E2, H1), jnp.bfloat16),
                   jax.ShapeDtypeStruct((2, H1), jnp.float32)],
    )(x, rm, b0)


def _p2_body(h_ref, st_ref, g_ref, b_ref, w2_ref, h2_ref, st2_ref):
    st = st_ref[...]
    mu = st[0:1] * (1.0 / E2)
    ex2 = st[1:2] * (1.0 / E2)
    inv = lax.rsqrt(ex2 - mu * mu + EPS)
    hv = h_ref[...].astype(jnp.float32)
    a = jnp.maximum((hv - mu) * (inv * g_ref[...]) + b_ref[...], 0.0)
    h2 = jnp.dot(a, w2_ref[...], preferred_element_type=jnp.float32)
    h2_ref[...] = h2.astype(jnp.bfloat16)
    st2 = jnp.concatenate([jnp.sum(h2, 0, keepdims=True),
                           jnp.sum(h2 * h2, 0, keepdims=True)], axis=0)

    @pl.when(pl.program_id(0) == 0)
    def _():
        st2_ref[...] = st2

    @pl.when(pl.program_id(0) != 0)
    def _():
        st2_ref[...] += st2


def _pass2(h, st, g1, b1, w2):
    return pl.pallas_call(
        _p2_body,
        grid=(NBLK,),
        in_specs=[pl.BlockSpec((BM, H1), lambda i: (i, 0)),
                  pl.BlockSpec((2, H1), lambda i: (0, 0)),
                  pl.BlockSpec((1, H1), lambda i: (0, 0)),
                  pl.BlockSpec((1, H1), lambda i: (0, 0)),
                  pl.BlockSpec((H1, C), lambda i: (0, 0))],
        out_specs=[pl.BlockSpec((BM, C), lambda i: (i, 0)),
                   pl.BlockSpec((2, C), lambda i: (0, 0))],
        out_shape=[jax.ShapeDtypeStruct((E2, C), jnp.bfloat16),
                   jax.ShapeDtypeStruct((2, C), jnp.float32)],
    )(h, st, g1, b1, w2)


def _p3_body(h2_ref, st2_ref, g_ref, b_ref, o_ref):
    st = st2_ref[...]
    mu = st[0:1] * (1.0 / E2)
    ex2 = st[1:2] * (1.0 / E2)
    inv = lax.rsqrt(ex2 - mu * mu + EPS)
    h2v = h2_ref[...].astype(jnp.float32)
    o_ref[...] = jnp.maximum((h2v - mu) * (inv * g_ref[...]) + b_ref[...], 0.0)


def _pass3(h2, st2, g2, b2):
    return pl.pallas_call(
        _p3_body,
        grid=(NBLK,),
        in_specs=[pl.BlockSpec((BM, C), lambda i: (i, 0)),
                  pl.BlockSpec((2, C), lambda i: (0, 0)),
                  pl.BlockSpec((1, C), lambda i: (0, 0)),
                  pl.BlockSpec((1, C), lambda i: (0, 0))],
        out_specs=pl.BlockSpec((BM, C), lambda i: (i, 0)),
        out_shape=jax.ShapeDtypeStruct((E2, C), jnp.float32),
    )(h2, st2, g2, b2)


def kernel(edge_rep, edge_index, W1, gamma1, beta1, W2, gamma2, beta2):
    src = edge_index[0]
    dst = edge_index[1]
    idx_own = jnp.stack([src, dst], axis=1).reshape(-1)
    idx_oth = jnp.stack([dst, src], axis=1).reshape(-1)
    idx2 = jnp.concatenate([idx_own, idx_oth])
    zeros_nc = jnp.zeros((N, C), jnp.float32)
    ones_nc = jnp.ones((SCH, C), jnp.float32)

    st, degw = _scatter(edge_rep, idx2, zeros_nc, ones_nc)
    s_acc = st[0]
    t_acc = st[1]

    p, q, b0 = _pq(W1, s_acc, t_acc, degw[0, :, :1], degw[1, :, :1])
    rm = _gather(p, q, idx_own, idx_oth)
    h, st1 = _pass1(edge_rep, rm, b0)
    h2, st2 = _pass2(h, st1, gamma1.reshape(1, H1), beta1.reshape(1, H1), W2)
    return _pass3(h2, st2, gamma2.reshape(1, C), beta2.reshape(1, C))


# speedup vs baseline: 1.3644x; 1.0003x over previous
import functools

import jax
import jax.numpy as jnp
from jax import lax
from jax.experimental import pallas as pl
from jax.experimental.pallas import tpu as pltpu
from jax.experimental.pallas import tpu_sc as plsc

N = 10000
C = 128
E2 = 320000
NC, NS = 2, 16
H1 = 256
EPS = 1e-5

ROWS_PER_TILE = E2 // NS
SCH = 80
NCHUNK = ROWS_PER_TILE // SCH
HALF = E2 // 2
HPT = HALF // NS
NCHUNK2 = HPT // SCH
NPT = 624
NREM = N - NS * NPT
NROFF = NS * NPT


def _scatter_body(edge_hbm, idx2_hbm, zeros_nc, ones_nc,
                  st_out, deg_out,
                  acc, ix0, ix1, rw0, rw1, ones_v,
                  sl0, sl1, sc0, sc1):
    ci = lax.axis_index("c")
    s = lax.axis_index("s")
    nb = s * NPT

    def zero_acc():
        pltpu.sync_copy(zeros_nc.at[pl.ds(nb, NPT)], acc.at[pl.ds(nb, NPT)])

        @pl.when(s == NS - 1)
        def _():
            pltpu.sync_copy(zeros_nc.at[pl.ds(NROFF, NREM)],
                            acc.at[pl.ds(NROFF, NREM)])

    def write_acc(out_hbm):
        pltpu.sync_copy(acc.at[pl.ds(nb, NPT)], out_hbm.at[ci, pl.ds(nb, NPT)])

        @pl.when(s == NS - 1)
        def _():
            pltpu.sync_copy(acc.at[pl.ds(NROFF, NREM)],
                            out_hbm.at[ci, pl.ds(NROFF, NREM)])

    zero_acc()
    pltpu.sync_copy(ones_nc, ones_v)
    plsc.subcore_barrier()

    slots = ((ix0, rw0, sl0, sc0), (ix1, rw1, sl1, sc1))

    def p1_load(k, b):
        ix, rw, sl, _ = slots[b]
        base = s * ROWS_PER_TILE + k * SCH
        pltpu.async_copy(idx2_hbm.at[pl.ds(ci * E2 + base, SCH)], ix, sl)
        pltpu.async_copy(edge_hbm.at[pl.ds(base, SCH)], rw, sl)

    def p1_load_wait(b):
        ix, rw, sl, _ = slots[b]
        pltpu.make_async_copy(idx2_hbm.at[pl.ds(0, SCH)], ix, sl).wait()
        pltpu.make_async_copy(edge_hbm.at[pl.ds(0, SCH)], rw, sl).wait()

    def scat(b, src):
        ix, rw, _, sc = slots[b]
        pltpu.async_copy(src if src is not None else rw, acc.at[ix], sc,
                         add=True)

    def scat_wait(b, src):
        ix, rw, _, sc = slots[b]
        pltpu.make_async_copy(src if src is not None else rw, acc.at[ix],
                              sc).wait()

    p1_load(0, 0)

    def p1_pair(j, carry):
        k0 = 2 * j
        p1_load_wait(0)
        scat(0, None)

        @pl.when(j > 0)
        def _():
            scat_wait(1, None)

        p1_load(k0 + 1, 1)
        p1_load_wait(1)
        scat(1, None)
        scat_wait(0, None)

        @pl.when(j < NCHUNK // 2 - 1)
        def _():
            p1_load(k0 + 2, 0)

        return carry

    lax.fori_loop(0, NCHUNK // 2, p1_pair, 0)
    scat_wait(1, None)
    plsc.subcore_barrier()
    write_acc(st_out)
    zero_acc()
    plsc.subcore_barrier()

    def p2_load(k, b):
        ix, _, sl, _ = slots[b]
        base = ci * HALF + s * HPT + k * SCH
        pltpu.async_copy(idx2_hbm.at[pl.ds(ci * E2 + base, SCH)], ix, sl)

    def p2_load_wait(b):
        ix, _, sl, _ = slots[b]
        pltpu.make_async_copy(idx2_hbm.at[pl.ds(0, SCH)], ix, sl).wait()

    p2_load(0, 0)
    p2_load_wait(0)
    scat(0, ones_v)
    p2_load(1, 1)

    def p2_pair(j, carry):
        k0 = 2 * j + 1
        p2_load_wait(1)
        scat(1, ones_v)
        scat_wait(0, ones_v)
        p2_load(k0 + 1, 0)
        p2_load_wait(0)
        scat(0, ones_v)
        scat_wait(1, ones_v)

        @pl.when(j < (NCHUNK2 - 1) // 2 - 1)
        def _():
            p2_load(k0 + 2, 1)

        return carry

    lax.fori_loop(0, (NCHUNK2 - 1) // 2, p2_pair, 0)
    scat_wait(0, ones_v)
    plsc.subcore_barrier()
    write_acc(deg_out)


_scatter = pl.kernel(
    _scatter_body,
    out_type=[jax.ShapeDtypeStruct((NC, N, C), jnp.float32),
              jax.ShapeDtypeStruct((NC, N, C), jnp.float32)],
    mesh=plsc.VectorSubcoreMesh(core_axis_name="c", subcore_axis_name="s",
                                num_cores=NC, num_subcores=NS),
    scratch_types=[pltpu.VMEM_SHARED((N, C), jnp.float32),
                   pltpu.VMEM((SCH,), jnp.int32),
                   pltpu.VMEM((SCH,), jnp.int32),
                   pltpu.VMEM((SCH, C), jnp.float32),
                   pltpu.VMEM((SCH, C), jnp.float32),
                   pltpu.VMEM((SCH, C), jnp.float32),
                   pltpu.SemaphoreType.DMA,
                   pltpu.SemaphoreType.DMA,
                   pltpu.SemaphoreType.DMA,
                   pltpu.SemaphoreType.DMA],
)

ROWS_PER_W = E2 // (NC * NS)
GCH = 40
NG = ROWS_PER_W // GCH


def _gather_body(p_hbm, q_hbm, idxs_hbm, idxt_hbm, r_out,
                 is0, it0, is1, it1, pb0, qb0, ob0, pb1, qb1, ob1,
                 si0, si1, sg0, sg1, ss0, ss1):
    ci = lax.axis_index("c")
    s = lax.axis_index("s")
    w = s * NC + ci
    rbase = w * ROWS_PER_W

    islot = ((is0, it0, si0), (is1, it1, si1))
    gslot = ((pb0, qb0, ob0, sg0, ss0), (pb1, qb1, ob1, sg1, ss1))

    def issue_idx(k, b):
        isv, itv, si = islot[b]
        pltpu.async_copy(idxs_hbm.at[pl.ds(rbase + k * GCH, GCH)], isv, si)
        pltpu.async_copy(idxt_hbm.at[pl.ds(rbase + k * GCH, GCH)], itv, si)

    def wait_idx(b):
        isv, itv, si = islot[b]
        pltpu.make_async_copy(idxs_hbm.at[pl.ds(0, GCH)], isv, si).wait()
        pltpu.make_async_copy(idxt_hbm.at[pl.ds(0, GCH)], itv, si).wait()

    def issue_gather(b):
        isv, itv, si = islot[b]
        pb, qb, ob, sg, ss = gslot[b]
        pltpu.async_copy(p_hbm.at[isv], pb, sg)
        pltpu.async_copy(q_hbm.at[itv], qb, sg)

    def wait_gather(b):
        isv, itv, si = islot[b]
        pb, qb, ob, sg, ss = gslot[b]
        pltpu.make_async_copy(p_hbm.at[isv], pb, sg).wait()
        pltpu.make_async_copy(q_hbm.at[itv], qb, sg).wait()

    def do_add(b):
        pb, qb, ob, sg, ss = gslot[b]

        def addrow(i, c2):
            for j in range(H1 // 16):
                sl = pl.ds(j * 16, 16)
                ob[i, sl] = pb[i, sl] + qb[i, sl]
            return c2

        lax.fori_loop(0, GCH, addrow, 0)

    def issue_store(k, b):
        pb, qb, ob, sg, ss = gslot[b]
        pltpu.async_copy(ob, r_out.at[pl.ds(rbase + k * GCH, GCH)], ss)

    def wait_store(b):
        pb, qb, ob, sg, ss = gslot[b]
        pltpu.make_async_copy(ob, r_out.at[pl.ds(rbase, GCH)], ss).wait()

    issue_idx(0, 0)
    wait_idx(0)
    issue_gather(0)
    issue_idx(1, 1)

    def pair(j, carry):
        k0 = 2 * j
        wait_idx(1)
        issue_gather(1)
        wait_gather(0)

        @pl.when(j > 0)
        def _():
            wait_store(0)

        do_add(0)
        issue_store(k0, 0)

        @pl.when(k0 + 2 < NG)
        def _():
            issue_idx(k0 + 2, 0)
            wait_idx(0)
            issue_gather(0)

        wait_gather(1)

        @pl.when(j > 0)
        def _():
            wait_store(1)

        do_add(1)
        issue_store(k0 + 1, 1)

        @pl.when(k0 + 3 < NG)
        def _():
            issue_idx(k0 + 3, 1)

        return carry

    lax.fori_loop(0, NG // 2, pair, 0)
    wait_store(0)
    wait_store(1)


_gather = pl.kernel(
    _gather_body,
    out_type=jax.ShapeDtypeStruct((E2, H1), jnp.float32),
    mesh=plsc.VectorSubcoreMesh(core_axis_name="c", subcore_axis_name="s",
                                num_cores=NC, num_subcores=NS),
    scratch_types=[pltpu.VMEM((GCH,), jnp.int32),
                   pltpu.VMEM((GCH,), jnp.int32),
                   pltpu.VMEM((GCH,), jnp.int32),
                   pltpu.VMEM((GCH,), jnp.int32),
                   pltpu.VMEM((GCH, H1), jnp.float32),
                   pltpu.VMEM((GCH, H1), jnp.float32),
                   pltpu.VMEM((GCH, H1), jnp.float32),
                   pltpu.VMEM((GCH, H1), jnp.float32),
                   pltpu.VMEM((GCH, H1), jnp.float32),
                   pltpu.VMEM((GCH, H1), jnp.float32),
                   pltpu.SemaphoreType.DMA,
                   pltpu.SemaphoreType.DMA,
                   pltpu.SemaphoreType.DMA,
                   pltpu.SemaphoreType.DMA,
                   pltpu.SemaphoreType.DMA,
                   pltpu.SemaphoreType.DMA],
)


def _pq_body(w1_ref, s_ref, t_ref, d0_ref, d1_ref, p_ref, q_ref, b0_ref):
    w = w1_ref[...]
    a0 = w[0:C]
    a1 = w[C:2 * C]
    a2 = w[2 * C:3 * C]
    a3 = w[3 * C:4 * C]
    a4 = w[4 * C:5 * C]
    b1 = a1 + a2 + a3 + a4
    b2 = a2 + a4
    b3 = a3 + a4
    sv = s_ref[...]
    d = d0_ref[...] + d1_ref[...]
    p_ref[...] = (jnp.dot(sv * d, b1, preferred_element_type=jnp.float32)
                  + jnp.dot(t_ref[...], b2, preferred_element_type=jnp.float32))
    q_ref[...] = jnp.dot(sv, b3, preferred_element_type=jnp.float32)
    b0_ref[...] = a0 + a4


def _pq(w1, s, t, d0, d1):
    return pl.pallas_call(
        _pq_body,
        out_shape=[jax.ShapeDtypeStruct((N, H1), jnp.float32),
                   jax.ShapeDtypeStruct((N, H1), jnp.float32),
                   jax.ShapeDtypeStruct((C, H1), jnp.float32)],
    )(w1, s, t, d0, d1)


BM = 2560
NBLK = E2 // BM


def _p1_body(x_ref, r_ref, b0_ref, h_ref, st_ref):
    h = (jnp.dot(x_ref[...], b0_ref[...], preferred_element_type=jnp.float32)
         + r_ref[...].astype(jnp.float32))
    h_ref[...] = h.astype(jnp.bfloat16)
    st = jnp.concatenate([jnp.sum(h, 0, keepdims=True),
                          jnp.sum(h * h, 0, keepdims=True)], axis=0)

    @pl.when(pl.program_id(0) == 0)
    def _():
        st_ref[...] = st

    @pl.when(pl.program_id(0) != 0)
    def _():
        st_ref[...] += st


def _pass1(x, rm, b0):
    return pl.pallas_call(
        _p1_body,
        grid=(NBLK,),
        in_specs=[pl.BlockSpec((BM, C), lambda i: (i, 0)),
                  pl.BlockSpec((BM, H1), lambda i: (i, 0)),
                  pl.BlockSpec((C, H1), lambda i: (0, 0))],
        out_specs=[pl.BlockSpec((BM, H1), lambda i: (i, 0)),
                   pl.BlockSpec((2, H1), lambda i: (0, 0))],
        out_shape=[jax.ShapeDtypeStruct((E2, H1), jnp.bfloat16),
                   jax.ShapeDtypeStruct((2, H1), jnp.float32)],
    )(x, rm, b0)


def _p2_body(h_ref, st_ref, g_ref, b_ref, w2_ref, h2_ref, st2_ref):
    st = st_ref[...]
    mu = st[0:1] * (1.0 / E2)
    ex2 = st[1:2] * (1.0 / E2)
    inv = lax.rsqrt(ex2 - mu * mu + EPS)
    hv = h_ref[...].astype(jnp.float32)
    a = jnp.maximum((hv - mu) * (inv * g_ref[...]) + b_ref[...], 0.0)
    h2 = jnp.dot(a, w2_ref[...], preferred_element_type=jnp.float32)
    h2_ref[...] = h2.astype(jnp.bfloat16)
    st2 = jnp.concatenate([jnp.sum(h2, 0, keepdims=True),
                           jnp.sum(h2 * h2, 0, keepdims=True)], axis=0)

    @pl.when(pl.program_id(0) == 0)
    def _():
        st2_ref[...] = st2

    @pl.when(pl.program_id(0) != 0)
    def _():
        st2_ref[...] += st2


def _pass2(h, st, g1, b1, w2):
    return pl.pallas_call(
        _p2_body,
        grid=(NBLK,),
        in_specs=[pl.BlockSpec((BM, H1), lambda i: (i, 0)),
                  pl.BlockSpec((2, H1), lambda i: (0, 0)),
                  pl.BlockSpec((1, H1), lambda i: (0, 0)),
                  pl.BlockSpec((1, H1), lambda i: (0, 0)),
                  pl.BlockSpec((H1, C), lambda i: (0, 0))],
        out_specs=[pl.BlockSpec((BM, C), lambda i: (i, 0)),
                   pl.BlockSpec((2, C), lambda i: (0, 0))],
        out_shape=[jax.ShapeDtypeStruct((E2, C), jnp.bfloat16),
                   jax.ShapeDtypeStruct((2, C), jnp.float32)],
    )(h, st, g1, b1, w2)


def _p3_body(h2_ref, st2_ref, g_ref, b_ref, o_ref):
    st = st2_ref[...]
    mu = st[0:1] * (1.0 / E2)
    ex2 = st[1:2] * (1.0 / E2)
    inv = lax.rsqrt(ex2 - mu * mu + EPS)
    h2v = h2_ref[...].astype(jnp.float32)
    o_ref[...] = jnp.maximum((h2v - mu) * (inv * g_ref[...]) + b_ref[...], 0.0)


def _pass3(h2, st2, g2, b2):
    return pl.pallas_call(
        _p3_body,
        grid=(NBLK,),
        in_specs=[pl.BlockSpec((BM, C), lambda i: (i, 0)),
                  pl.BlockSpec((2, C), lambda i: (0, 0)),
                  pl.BlockSpec((1, C), lambda i: (0, 0)),
                  pl.BlockSpec((1, C), lambda i: (0, 0))],
        out_specs=pl.BlockSpec((BM, C), lambda i: (i, 0)),
        out_shape=jax.ShapeDtypeStruct((E2, C), jnp.float32),
    )(h2, st2, g2, b2)


def kernel(edge_rep, edge_index, W1, gamma1, beta1, W2, gamma2, beta2):
    src = edge_index[0]
    dst = edge_index[1]
    idx_own = jnp.stack([src, dst], axis=1).reshape(-1)
    idx_oth = jnp.stack([dst, src], axis=1).reshape(-1)
    idx2 = jnp.concatenate([idx_own, idx_oth])
    zeros_nc = jnp.zeros((N, C), jnp.float32)
    ones_nc = jnp.ones((SCH, C), jnp.float32)

    st, degw = _scatter(edge_rep, idx2, zeros_nc, ones_nc)
    s_acc = st[0]
    t_acc = st[1]

    p, q, b0 = _pq(W1, s_acc, t_acc, degw[0, :, :1], degw[1, :, :1])
    rm = _gather(p, q, idx_own, idx_oth)
    h, st1 = _pass1(edge_rep, rm, b0)
    h2, st2 = _pass2(h, st1, gamma1.reshape(1, H1), beta1.reshape(1, H1), W2)
    return _pass3(h2, st2, gamma2.reshape(1, C), beta2.reshape(1, C))


# speedup vs baseline: 1.3651x; 1.0005x over previous
import jax
import jax.numpy as jnp
from jax import lax
from jax.experimental import pallas as pl
from jax.experimental.pallas import tpu as pltpu
from jax.experimental.pallas import tpu_sc as plsc

N = 10000
C = 128
E2 = 320000
NC, NS = 2, 16
H1 = 256
EPS = 1e-5

ROWS_PER_TILE = E2 // NS
SCH = 80
NCHUNK = ROWS_PER_TILE // SCH
HALF = E2 // 2
HPT = HALF // NS
NCHUNK2 = HPT // SCH
NPT = 624
NREM = N - NS * NPT
NROFF = NS * NPT


def _scatter_body(edge_hbm, idx2_hbm, zeros_nc, ones_nc,
                  st_out, deg_out,
                  acc, ix0, ix1, rw0, rw1, ones_v,
                  sl0, sl1, sc0, sc1):
    ci = lax.axis_index("c")
    s = lax.axis_index("s")
    nb = s * NPT

    def zero_acc():
        pltpu.sync_copy(zeros_nc.at[pl.ds(nb, NPT)], acc.at[pl.ds(nb, NPT)])

        @pl.when(s == NS - 1)
        def _():
            pltpu.sync_copy(zeros_nc.at[pl.ds(NROFF, NREM)],
                            acc.at[pl.ds(NROFF, NREM)])

    def write_acc(out_hbm):
        pltpu.sync_copy(acc.at[pl.ds(nb, NPT)], out_hbm.at[ci, pl.ds(nb, NPT)])

        @pl.when(s == NS - 1)
        def _():
            pltpu.sync_copy(acc.at[pl.ds(NROFF, NREM)],
                            out_hbm.at[ci, pl.ds(NROFF, NREM)])

    zero_acc()
    pltpu.sync_copy(ones_nc, ones_v)
    plsc.subcore_barrier()

    slots = ((ix0, rw0, sl0, sc0), (ix1, rw1, sl1, sc1))

    def p1_load(k, b):
        ix, rw, sl, _ = slots[b]
        base = s * ROWS_PER_TILE + k * SCH
        pltpu.async_copy(idx2_hbm.at[pl.ds(ci * E2 + base, SCH)], ix, sl)
        pltpu.async_copy(edge_hbm.at[pl.ds(base, SCH)], rw, sl)

    def p1_load_wait(b):
        ix, rw, sl, _ = slots[b]
        pltpu.make_async_copy(idx2_hbm.at[pl.ds(0, SCH)], ix, sl).wait()
        pltpu.make_async_copy(edge_hbm.at[pl.ds(0, SCH)], rw, sl).wait()

    def scat(b, src):
        ix, rw, _, sc = slots[b]
        pltpu.async_copy(src if src is not None else rw, acc.at[ix], sc,
                         add=True)

    def scat_wait(b, src):
        ix, rw, _, sc = slots[b]
        pltpu.make_async_copy(src if src is not None else rw, acc.at[ix],
                              sc).wait()

    p1_load(0, 0)

    def p1_pair(j, carry):
        k0 = 2 * j
        p1_load_wait(0)
        scat(0, None)

        @pl.when(j > 0)
        def _():
            scat_wait(1, None)

        p1_load(k0 + 1, 1)
        p1_load_wait(1)
        scat(1, None)
        scat_wait(0, None)

        @pl.when(j < NCHUNK // 2 - 1)
        def _():
            p1_load(k0 + 2, 0)

        return carry

    lax.fori_loop(0, NCHUNK // 2, p1_pair, 0)
    scat_wait(1, None)
    plsc.subcore_barrier()
    write_acc(st_out)
    zero_acc()
    plsc.subcore_barrier()

    def p2_load(k, b):
        ix, _, sl, _ = slots[b]
        base = ci * HALF + s * HPT + k * SCH
        pltpu.async_copy(idx2_hbm.at[pl.ds(ci * E2 + base, SCH)], ix, sl)

    def p2_load_wait(b):
        ix, _, sl, _ = slots[b]
        pltpu.make_async_copy(idx2_hbm.at[pl.ds(0, SCH)], ix, sl).wait()

    p2_load(0, 0)
    p2_load_wait(0)
    scat(0, ones_v)
    p2_load(1, 1)

    def p2_pair(j, carry):
        k0 = 2 * j + 1
        p2_load_wait(1)
        scat(1, ones_v)
        scat_wait(0, ones_v)
        p2_load(k0 + 1, 0)
        p2_load_wait(0)
        scat(0, ones_v)
        scat_wait(1, ones_v)

        @pl.when(j < (NCHUNK2 - 1) // 2 - 1)
        def _():
            p2_load(k0 + 2, 1)

        return carry

    lax.fori_loop(0, (NCHUNK2 - 1) // 2, p2_pair, 0)
    scat_wait(0, ones_v)
    plsc.subcore_barrier()
    write_acc(deg_out)


_scatter = pl.kernel(
    _scatter_body,
    out_type=[jax.ShapeDtypeStruct((NC, N, C), jnp.float32),
              jax.ShapeDtypeStruct((NC, N, C), jnp.float32)],
    mesh=plsc.VectorSubcoreMesh(core_axis_name="c", subcore_axis_name="s",
                                num_cores=NC, num_subcores=NS),
    scratch_types=[pltpu.VMEM_SHARED((N, C), jnp.float32),
                   pltpu.VMEM((SCH,), jnp.int32),
                   pltpu.VMEM((SCH,), jnp.int32),
                   pltpu.VMEM((SCH, C), jnp.float32),
                   pltpu.VMEM((SCH, C), jnp.float32),
                   pltpu.VMEM((SCH, C), jnp.float32),
                   pltpu.SemaphoreType.DMA,
                   pltpu.SemaphoreType.DMA,
                   pltpu.SemaphoreType.DMA,
                   pltpu.SemaphoreType.DMA],
)

ROWS_PER_W = E2 // (NC * NS)
GCH = 40
NG = ROWS_PER_W // GCH


def _gather_body(p_hbm, q_hbm, idxs_hbm, idxt_hbm, r_out,
                 is0, it0, is1, it1, pb0, qb0, ob0, pb1, qb1, ob1,
                 si0, si1, sg0, sg1, ss0, ss1):
    ci = lax.axis_index("c")
    s = lax.axis_index("s")
    w = s * NC + ci
    rbase = w * ROWS_PER_W

    islot = ((is0, it0, si0), (is1, it1, si1))
    gslot = ((pb0, qb0, ob0, sg0, ss0), (pb1, qb1, ob1, sg1, ss1))

    def issue_idx(k, b):
        isv, itv, si = islot[b]
        pltpu.async_copy(idxs_hbm.at[pl.ds(rbase + k * GCH, GCH)], isv, si)
        pltpu.async_copy(idxt_hbm.at[pl.ds(rbase + k * GCH, GCH)], itv, si)

    def wait_idx(b):
        isv, itv, si = islot[b]
        pltpu.make_async_copy(idxs_hbm.at[pl.ds(0, GCH)], isv, si).wait()
        pltpu.make_async_copy(idxt_hbm.at[pl.ds(0, GCH)], itv, si).wait()

    def issue_gather(b):
        isv, itv, si = islot[b]
        pb, qb, ob, sg, ss = gslot[b]
        pltpu.async_copy(p_hbm.at[isv], pb, sg)
        pltpu.async_copy(q_hbm.at[itv], qb, sg)

    def wait_gather(b):
        isv, itv, si = islot[b]
        pb, qb, ob, sg, ss = gslot[b]
        pltpu.make_async_copy(p_hbm.at[isv], pb, sg).wait()
        pltpu.make_async_copy(q_hbm.at[itv], qb, sg).wait()

    def do_add(b):
        pb, qb, ob, sg, ss = gslot[b]

        def addrow(i, c2):
            for j in range(H1 // 16):
                sl = pl.ds(j * 16, 16)
                ob[i, sl] = pb[i, sl] + qb[i, sl]
            return c2

        lax.fori_loop(0, GCH, addrow, 0)

    def issue_store(k, b):
        pb, qb, ob, sg, ss = gslot[b]
        pltpu.async_copy(ob, r_out.at[pl.ds(rbase + k * GCH, GCH)], ss)

    def wait_store(b):
        pb, qb, ob, sg, ss = gslot[b]
        pltpu.make_async_copy(ob, r_out.at[pl.ds(rbase, GCH)], ss).wait()

    issue_idx(0, 0)
    wait_idx(0)
    issue_gather(0)
    issue_idx(1, 1)

    def pair(j, carry):
        k0 = 2 * j
        wait_idx(1)
        issue_gather(1)
        wait_gather(0)

        @pl.when(j > 0)
        def _():
            wait_store(0)

        do_add(0)
        issue_store(k0, 0)

        @pl.when(k0 + 2 < NG)
        def _():
            issue_idx(k0 + 2, 0)
            wait_idx(0)
            issue_gather(0)

        wait_gather(1)

        @pl.when(j > 0)
        def _():
            wait_store(1)

        do_add(1)
        issue_store(k0 + 1, 1)

        @pl.when(k0 + 3 < NG)
        def _():
            issue_idx(k0 + 3, 1)

        return carry

    lax.fori_loop(0, NG // 2, pair, 0)
    wait_store(0)
    wait_store(1)


_gather = pl.kernel(
    _gather_body,
    out_type=jax.ShapeDtypeStruct((E2, H1), jnp.float32),
    mesh=plsc.VectorSubcoreMesh(core_axis_name="c", subcore_axis_name="s",
                                num_cores=NC, num_subcores=NS),
    scratch_types=[pltpu.VMEM((GCH,), jnp.int32),
                   pltpu.VMEM((GCH,), jnp.int32),
                   pltpu.VMEM((GCH,), jnp.int32),
                   pltpu.VMEM((GCH,), jnp.int32),
                   pltpu.VMEM((GCH, H1), jnp.float32),
                   pltpu.VMEM((GCH, H1), jnp.float32),
                   pltpu.VMEM((GCH, H1), jnp.float32),
                   pltpu.VMEM((GCH, H1), jnp.float32),
                   pltpu.VMEM((GCH, H1), jnp.float32),
                   pltpu.VMEM((GCH, H1), jnp.float32),
                   pltpu.SemaphoreType.DMA,
                   pltpu.SemaphoreType.DMA,
                   pltpu.SemaphoreType.DMA,
                   pltpu.SemaphoreType.DMA,
                   pltpu.SemaphoreType.DMA,
                   pltpu.SemaphoreType.DMA],
)


def _pq_body(w1_ref, s_ref, t_ref, d0_ref, d1_ref, p_ref, q_ref, b0_ref):
    w = w1_ref[...]
    a0 = w[0:C]
    a1 = w[C:2 * C]
    a2 = w[2 * C:3 * C]
    a3 = w[3 * C:4 * C]
    a4 = w[4 * C:5 * C]
    b1 = a1 + a2 + a3 + a4
    b2 = a2 + a4
    b3 = a3 + a4
    sv = s_ref[...]
    d = d0_ref[...] + d1_ref[...]
    p_ref[...] = (jnp.dot(sv * d, b1, preferred_element_type=jnp.float32)
                  + jnp.dot(t_ref[...], b2, preferred_element_type=jnp.float32))
    q_ref[...] = jnp.dot(sv, b3, preferred_element_type=jnp.float32)
    b0_ref[...] = a0 + a4


def _pq(w1, s, t, d0, d1):
    return pl.pallas_call(
        _pq_body,
        out_shape=[jax.ShapeDtypeStruct((N, H1), jnp.float32),
                   jax.ShapeDtypeStruct((N, H1), jnp.float32),
                   jax.ShapeDtypeStruct((C, H1), jnp.float32)],
    )(w1, s, t, d0, d1)


BM = 2560
NBLK = E2 // BM


def _p1_body(x_ref, r_ref, b0_ref, h_ref, st_ref):
    h = (jnp.dot(x_ref[...], b0_ref[...], preferred_element_type=jnp.float32)
         + r_ref[...].astype(jnp.float32))
    h_ref[...] = h.astype(jnp.bfloat16)
    st = jnp.concatenate([jnp.sum(h, 0, keepdims=True),
                          jnp.sum(h * h, 0, keepdims=True)], axis=0)

    @pl.when(pl.program_id(0) == 0)
    def _():
        st_ref[...] = st

    @pl.when(pl.program_id(0) != 0)
    def _():
        st_ref[...] += st


def _pass1(x, rm, b0):
    return pl.pallas_call(
        _p1_body,
        grid=(NBLK,),
        in_specs=[pl.BlockSpec((BM, C), lambda i: (i, 0)),
                  pl.BlockSpec((BM, H1), lambda i: (i, 0)),
                  pl.BlockSpec((C, H1), lambda i: (0, 0))],
        out_specs=[pl.BlockSpec((BM, H1), lambda i: (i, 0)),
                   pl.BlockSpec((2, H1), lambda i: (0, 0))],
        out_shape=[jax.ShapeDtypeStruct((E2, H1), jnp.bfloat16),
                   jax.ShapeDtypeStruct((2, H1), jnp.float32)],
    )(x, rm, b0)


def _p2_body(h_ref, st_ref, g_ref, b_ref, w2_ref, h2_ref, st2_ref):
    st = st_ref[...]
    mu = st[0:1] * (1.0 / E2)
    ex2 = st[1:2] * (1.0 / E2)
    inv = lax.rsqrt(ex2 - mu * mu + EPS)
    hv = h_ref[...].astype(jnp.float32)
    a = jnp.maximum((hv - mu) * (inv * g_ref[...]) + b_ref[...], 0.0)
    h2 = jnp.dot(a, w2_ref[...], preferred_element_type=jnp.float32)
    h2_ref[...] = h2.astype(jnp.bfloat16)
    st2 = jnp.concatenate([jnp.sum(h2, 0, keepdims=True),
                           jnp.sum(h2 * h2, 0, keepdims=True)], axis=0)

    @pl.when(pl.program_id(0) == 0)
    def _():
        st2_ref[...] = st2

    @pl.when(pl.program_id(0) != 0)
    def _():
        st2_ref[...] += st2


def _pass2(h, st, g1, b1, w2):
    return pl.pallas_call(
        _p2_body,
        grid=(NBLK,),
        in_specs=[pl.BlockSpec((BM, H1), lambda i: (i, 0)),
                  pl.BlockSpec((2, H1), lambda i: (0, 0)),
                  pl.BlockSpec((1, H1), lambda i: (0, 0)),
                  pl.BlockSpec((1, H1), lambda i: (0, 0)),
                  pl.BlockSpec((H1, C), lambda i: (0, 0))],
        out_specs=[pl.BlockSpec((BM, C), lambda i: (i, 0)),
                   pl.BlockSpec((2, C), lambda i: (0, 0))],
        out_shape=[jax.ShapeDtypeStruct((E2, C), jnp.bfloat16),
                   jax.ShapeDtypeStruct((2, C), jnp.float32)],
    )(h, st, g1, b1, w2)


def _p3_body(h2_ref, st2_ref, g_ref, b_ref, o_ref):
    st = st2_ref[...]
    mu = st[0:1] * (1.0 / E2)
    ex2 = st[1:2] * (1.0 / E2)
    inv = lax.rsqrt(ex2 - mu * mu + EPS)
    h2v = h2_ref[...].astype(jnp.float32)
    o_ref[...] = jnp.maximum((h2v - mu) * (inv * g_ref[...]) + b_ref[...], 0.0)


def _pass3(h2, st2, g2, b2):
    return pl.pallas_call(
        _p3_body,
        grid=(NBLK,),
        in_specs=[pl.BlockSpec((BM, C), lambda i: (i, 0)),
                  pl.BlockSpec((2, C), lambda i: (0, 0)),
                  pl.BlockSpec((1, C), lambda i: (0, 0)),
                  pl.BlockSpec((1, C), lambda i: (0, 0))],
        out_specs=pl.BlockSpec((BM, C), lambda i: (i, 0)),
        out_shape=jax.ShapeDtypeStruct((E2, C), jnp.float32),
    )(h2, st2, g2, b2)


def kernel(edge_rep, edge_index, W1, gamma1, beta1, W2, gamma2, beta2):
    src = edge_index[0]
    dst = edge_index[1]
    idx_own = jnp.stack([src, dst], axis=1).reshape(-1)
    idx_oth = jnp.stack([dst, src], axis=1).reshape(-1)
    idx2 = jnp.concatenate([idx_own, idx_oth])
    zeros_nc = jnp.zeros((N, C), jnp.float32)
    ones_nc = jnp.ones((SCH, C), jnp.float32)

    st, degw = _scatter(edge_rep, idx2, zeros_nc, ones_nc)
    s_acc = st[0]
    t_acc = st[1]

    p, q, b0 = _pq(W1, s_acc, t_acc, degw[0, :, :1], degw[1, :, :1])
    rm = _gather(p, q, idx_own, idx_oth)
    h, st1 = _pass1(edge_rep, rm, b0)
    h2, st2 = _pass2(h, st1, gamma1.reshape(1, H1), beta1.reshape(1, H1), W2)
    return _pass3(h2, st2, gamma2.reshape(1, C), beta2.reshape(1, C))
